# baseline (speedup 1.0000x reference)
import functools

import jax
import jax.numpy as jnp
from jax import lax
from jax.experimental import pallas as pl
from jax.experimental.pallas import tpu as pltpu
from jax.experimental.pallas import tpu_sc as plsc

N = 10000
E = 320000
D = 128
H1 = 64
H2 = 32
G = 64

CH = 128
NC = 2
NS = 16
NW = NC * NS
CPW = 80
NCHP = NW * CPW
E_PAD = NCHP * CH

_SC_PARAMS = pltpu.CompilerParams(
    needs_layout_passes=False, use_tc_tiling_on_sc=False)

STRIPE = 624
LAST_STRIPE = N - (NS - 1) * STRIPE


def _zero_rows(buf, nrows, width):
    def body(k, _):
        for p in range(width // 16):
            buf[k, pl.ds(p * 16, 16)] = jnp.zeros((16,), jnp.float32)
        return 0
    lax.fori_loop(0, nrows, body, 0)


def _zero_acc_stripe(acc, zsrc, s):
    base = pl.multiple_of(s * STRIPE, 8)

    @pl.when(s < NS - 1)
    def _():
        for j, sz in enumerate([128, 128, 128, 128, 112]):
            pltpu.sync_copy(
                zsrc.at[pl.ds(0, sz)],
                acc.at[pl.ds(pl.multiple_of(base + j * 128, 8), sz)])

    @pl.when(s == NS - 1)
    def _():
        for j in range(5):
            pltpu.sync_copy(
                zsrc,
                acc.at[pl.ds(pl.multiple_of(base + j * 128, 8), 128)])


def _copy_out_stripe(acc, out_hbm, c, s):
    base = pl.multiple_of(s * STRIPE, 8)

    @pl.when(s < NS - 1)
    def _():
        pltpu.sync_copy(acc.at[pl.ds(base, STRIPE)],
                        out_hbm.at[c, pl.ds(base, STRIPE)])

    @pl.when(s == NS - 1)
    def _():
        pltpu.sync_copy(acc.at[pl.ds(base, LAST_STRIPE)],
                        out_hbm.at[c, pl.ds(base, LAST_STRIPE)])


def _make_deg_kernel():
    mesh = plsc.VectorSubcoreMesh(core_axis_name="c", subcore_axis_name="s")

    @functools.partial(
        pl.kernel,
        out_type=jax.ShapeDtypeStruct((NC, N), jnp.float32),
        mesh=mesh,
        compiler_params=_SC_PARAMS,
        scratch_types=[
            pltpu.VMEM((CPW, CH), jnp.int32),
            pltpu.VMEM((CPW, CH), jnp.float32),
            pltpu.VMEM((LAST_STRIPE,), jnp.float32),
            pltpu.VMEM_SHARED((N,), jnp.float32),
            pltpu.SemaphoreType.DMA,
        ],
    )
    def k(col_hbm, ew_hbm, out_hbm, col_all, ew_all, zbuf, acc, sem):
        c = lax.axis_index("c")
        s = lax.axis_index("s")
        w = c * NS + s

        def zb(k2, _):
            zbuf[pl.ds(k2 * 16, 16)] = jnp.zeros((16,), jnp.float32)
            return 0
        lax.fori_loop(0, LAST_STRIPE // 16, zb, 0)
        base = pl.multiple_of(s * STRIPE, 8)

        @pl.when(s < NS - 1)
        def _():
            pltpu.sync_copy(zbuf.at[pl.ds(0, STRIPE)],
                            acc.at[pl.ds(base, STRIPE)])

        @pl.when(s == NS - 1)
        def _():
            pltpu.sync_copy(zbuf, acc.at[pl.ds(base, LAST_STRIPE)])

        plsc.subcore_barrier()
        cb = pl.multiple_of(w * CPW, 8)
        pltpu.sync_copy(col_hbm.at[pl.ds(cb, CPW)], col_all)
        pltpu.sync_copy(ew_hbm.at[pl.ds(cb, CPW)], ew_all)

        def grp(i, _):
            for j in range(8):
                g = i * 8 + j
                pltpu.async_copy(ew_all.at[g], acc.at[col_all.at[g]], sem,
                                 add=True)
            for j in range(8):
                g = i * 8 + j
                pltpu.make_async_copy(ew_all.at[g], acc.at[col_all.at[g]],
                                      sem).wait()
            return 0

        lax.fori_loop(0, CPW // 8, grp, 0)
        plsc.subcore_barrier()

        @pl.when(s < NS - 1)
        def _():
            pltpu.sync_copy(acc.at[pl.ds(base, STRIPE)],
                            out_hbm.at[c, pl.ds(base, STRIPE)])

        @pl.when(s == NS - 1)
        def _():
            pltpu.sync_copy(acc.at[pl.ds(base, LAST_STRIPE)],
                            out_hbm.at[c, pl.ds(base, LAST_STRIPE)])

    return k


def _make_conv_kernel(width):
    mesh = plsc.VectorSubcoreMesh(core_axis_name="c", subcore_axis_name="s")

    @functools.partial(
        pl.kernel,
        out_type=jax.ShapeDtypeStruct((NC, N, width), jnp.float32),
        mesh=mesh,
        compiler_params=_SC_PARAMS,
        scratch_types=[
            pltpu.VMEM((CPW, CH), jnp.int32),
            pltpu.VMEM((CPW, CH), jnp.int32),
            pltpu.VMEM((CPW, CH), jnp.float32),
            pltpu.VMEM((CH, width), jnp.float32),
            pltpu.VMEM((CH, width), jnp.float32),
            pltpu.VMEM_SHARED((N, width), jnp.float32),
            pltpu.VMEM_SHARED((N, width), jnp.float32),
            pltpu.SemaphoreType.DMA,
            pltpu.SemaphoreType.DMA,
            pltpu.SemaphoreType.DMA,
            pltpu.SemaphoreType.DMA,
        ],
    )
    def k(row_hbm, col_hbm, ew_hbm, xws_hbm, out_hbm,
          row_all, col_all, ew_all, rows0, rows1, acc, xws_s, g0, g1, s0, s1):
        c = lax.axis_index("c")
        s = lax.axis_index("s")
        w = c * NS + s
        base = pl.multiple_of(s * STRIPE, 8)

        @pl.when(s < NS - 1)
        def _():
            pltpu.sync_copy(xws_hbm.at[pl.ds(base, STRIPE)],
                            xws_s.at[pl.ds(base, STRIPE)])

        @pl.when(s == NS - 1)
        def _():
            pltpu.sync_copy(xws_hbm.at[pl.ds(base, LAST_STRIPE)],
                            xws_s.at[pl.ds(base, LAST_STRIPE)])

        _zero_rows(rows0, CH, width)
        _zero_acc_stripe(acc, rows0, s)
        plsc.subcore_barrier()
        cb = pl.multiple_of(w * CPW, 8)
        pltpu.sync_copy(row_hbm.at[pl.ds(cb, CPW)], row_all)
        pltpu.sync_copy(col_hbm.at[pl.ds(cb, CPW)], col_all)
        pltpu.sync_copy(ew_hbm.at[pl.ds(cb, CPW)], ew_all)
        pltpu.async_copy(xws_s.at[row_all.at[0]], rows0, g0)

        def pair(i, _):
            for j in range(2):
                g = i * 2 + j
                if j == 0:
                    cur, gc, sc_ = rows0, g0, s0
                    nxt, gn, sn_ = rows1, g1, s1
                else:
                    cur, gc, sc_ = rows1, g1, s1
                    nxt, gn, sn_ = rows0, g0, s0
                pltpu.make_async_copy(xws_s.at[row_all.at[g]], cur,
                                      gc).wait()

                @pl.when(g >= 1)
                def _():
                    pltpu.make_async_copy(nxt, acc.at[col_all.at[g - 1]],
                                          sn_).wait()

                @pl.when(g + 1 < CPW)
                def _():
                    pltpu.async_copy(xws_s.at[row_all.at[g + 1]], nxt, gn)

                @plsc.parallel_loop(0, CH, step=1, unroll=16)
                def _(k2):
                    f = plsc.load_gather(
                        ew_all, [jnp.full((16,), g, jnp.int32),
                                 jnp.full((16,), k2, jnp.int32)])
                    for p in range(width // 16):
                        v = cur[k2, pl.ds(p * 16, 16)]
                        cur[k2, pl.ds(p * 16, 16)] = v * f
                pltpu.async_copy(cur, acc.at[col_all.at[g]], sc_, add=True)
            return 0

        lax.fori_loop(0, CPW // 2, pair, 0)
        pltpu.make_async_copy(rows1, acc.at[col_all.at[CPW - 1]], s1).wait()
        plsc.subcore_barrier()
        _copy_out_stripe(acc, out_hbm, c, s)

    return k


_deg_kernel = _make_deg_kernel()
_conv64 = _make_conv_kernel(H1)
_conv32 = _make_conv_kernel(H2)


def _tc1_body(degp_ref, x_ref, w1_ref, dinv_ref, xws1_ref):
    deg = 1.0 + degp_ref[0] + degp_ref[1]
    dinv = lax.rsqrt(deg)
    dinv_ref[...] = dinv
    xw = jnp.dot(x_ref[...], w1_ref[...], preferred_element_type=jnp.float32)
    xws1_ref[...] = xw * dinv


def _tc2_body(s1_ref, xws1_ref, dinv_ref, b1_ref, w2_ref, xws2_ref):
    dinv = dinv_ref[...]
    pre = dinv * (s1_ref[0] + s1_ref[1] + xws1_ref[...]) + b1_ref[...]
    h1 = jnp.maximum(pre, 0.0)
    xw2 = jnp.dot(h1, w2_ref[...], preferred_element_type=jnp.float32)
    xws2_ref[...] = xw2 * dinv


WIN = 1000
NWIN = N // WIN


def _tc3_body(s2_ref, xws2_ref, dinv_ref, b2_ref, batch_ref, bnds_ref,
              wl_ref, bl_ref, out_ref, pooled):
    dinv = dinv_ref[...]
    pre = dinv * (s2_ref[0] + s2_ref[1] + xws2_ref[...]) + b2_ref[...]
    h2 = jnp.maximum(pre, 0.0)
    batch = batch_ref[...]
    pooled[...] = jnp.full((G, H2), -jnp.inf, jnp.float32)
    for wdw in range(NWIN):
        rows = h2[wdw * WIN:(wdw + 1) * WIN]
        bwin = batch[wdw * WIN:(wdw + 1) * WIN]
        g0 = bnds_ref[wdw]
        g1 = bnds_ref[NWIN + wdw]

        def seg(g, _):
            m = bwin == g
            v = jnp.where(m, rows, -jnp.inf)
            cur = pooled[pl.ds(g, 1), :]
            pooled[pl.ds(g, 1), :] = jnp.maximum(
                cur, jnp.max(v, axis=0, keepdims=True))
            return 0

        lax.fori_loop(g0, g1 + 1, seg, 0)
    out_ref[...] = jnp.dot(pooled[...], wl_ref[...],
                           preferred_element_type=jnp.float32) + bl_ref[...]


def kernel(x, edge_index, edge_weight, batch, W1, b1, W2, b2, Wl, bl):
    pad = E_PAD - E
    idt = edge_index.dtype
    row2d = jnp.concatenate(
        [edge_index[0], jnp.zeros((pad,), idt)]).reshape(NCHP, CH)
    col2d = jnp.concatenate(
        [edge_index[1], jnp.zeros((pad,), idt)]).reshape(NCHP, CH)
    ew2d = jnp.concatenate(
        [edge_weight, jnp.zeros((pad,), edge_weight.dtype)]).reshape(NCHP, CH)

    degp = _deg_kernel(col2d, ew2d).reshape(NC, N, 1)

    dinv, xws1 = pl.pallas_call(
        _tc1_body,
        out_shape=(jax.ShapeDtypeStruct((N, 1), jnp.float32),
                   jax.ShapeDtypeStruct((N, H1), jnp.float32)),
    )(degp, x, W1)

    s1 = _conv64(row2d, col2d, ew2d, xws1)

    xws2 = pl.pallas_call(
        _tc2_body,
        out_shape=jax.ShapeDtypeStruct((N, H2), jnp.float32),
    )(s1, xws1, dinv, b1.reshape(1, H1), W2)

    s2 = _conv32(row2d, col2d, ew2d, xws2)

    bnds = jnp.concatenate([batch[0::WIN], batch[WIN - 1::WIN]])

    out = pl.pallas_call(
        _tc3_body,
        out_shape=jax.ShapeDtypeStruct((G, 4), jnp.float32),
        in_specs=[
            pl.BlockSpec(), pl.BlockSpec(), pl.BlockSpec(), pl.BlockSpec(),
            pl.BlockSpec(), pl.BlockSpec(memory_space=pltpu.SMEM),
            pl.BlockSpec(), pl.BlockSpec()],
        scratch_shapes=[pltpu.VMEM((G, H2), jnp.float32)],
    )(s2, xws2, dinv, b2.reshape(1, H2), batch.reshape(N, 1),
      bnds.astype(jnp.int32), Wl, bl.reshape(1, 4))

    return out


# speedup vs baseline: 1.0203x; 1.0203x over previous
import functools

import jax
import jax.numpy as jnp
from jax import lax
from jax.experimental import pallas as pl
from jax.experimental.pallas import tpu as pltpu
from jax.experimental.pallas import tpu_sc as plsc

N = 10000
E = 320000
D = 128
H1 = 64
H2 = 32
G = 64

CH = 128
NC = 2
NS = 16
NW = NC * NS
CPW = 80
CPW0 = 88
CPW1 = 72
NCHP = NW * CPW
E_PAD = NCHP * CH

_SC_PARAMS = pltpu.CompilerParams(
    needs_layout_passes=False, use_tc_tiling_on_sc=False)

STRIPE = 624
LAST_STRIPE = N - (NS - 1) * STRIPE


def _zero_rows(buf, nrows, width):
    def body(k, _):
        for p in range(width // 16):
            buf[k, pl.ds(p * 16, 16)] = jnp.zeros((16,), jnp.float32)
        return 0
    lax.fori_loop(0, nrows, body, 0)


def _zero_acc_stripe(acc, zsrc, s):
    base = pl.multiple_of(s * STRIPE, 8)

    @pl.when(s < NS - 1)
    def _():
        for j, sz in enumerate([128, 128, 128, 128, 112]):
            pltpu.sync_copy(
                zsrc.at[pl.ds(0, sz)],
                acc.at[pl.ds(pl.multiple_of(base + j * 128, 8), sz)])

    @pl.when(s == NS - 1)
    def _():
        for j in range(5):
            pltpu.sync_copy(
                zsrc,
                acc.at[pl.ds(pl.multiple_of(base + j * 128, 8), 128)])


def _copy_out_stripe(acc, out_hbm, c, s):
    base = pl.multiple_of(s * STRIPE, 8)

    @pl.when(s < NS - 1)
    def _():
        pltpu.sync_copy(acc.at[pl.ds(base, STRIPE)],
                        out_hbm.at[c, pl.ds(base, STRIPE)])

    @pl.when(s == NS - 1)
    def _():
        pltpu.sync_copy(acc.at[pl.ds(base, LAST_STRIPE)],
                        out_hbm.at[c, pl.ds(base, LAST_STRIPE)])


def _make_deg_kernel():
    mesh = plsc.VectorSubcoreMesh(core_axis_name="c", subcore_axis_name="s")

    @functools.partial(
        pl.kernel,
        out_type=jax.ShapeDtypeStruct((NC, N), jnp.float32),
        mesh=mesh,
        compiler_params=_SC_PARAMS,
        scratch_types=[
            pltpu.VMEM((CPW, CH), jnp.int32),
            pltpu.VMEM((CPW, CH), jnp.float32),
            pltpu.VMEM((LAST_STRIPE,), jnp.float32),
            pltpu.VMEM_SHARED((N,), jnp.float32),
            pltpu.SemaphoreType.DMA,
        ],
    )
    def k(col_hbm, ew_hbm, out_hbm, col_all, ew_all, zbuf, acc, sem):
        c = lax.axis_index("c")
        s = lax.axis_index("s")
        w = c * NS + s

        def zb(k2, _):
            zbuf[pl.ds(k2 * 16, 16)] = jnp.zeros((16,), jnp.float32)
            return 0
        lax.fori_loop(0, LAST_STRIPE // 16, zb, 0)
        base = pl.multiple_of(s * STRIPE, 8)

        @pl.when(s < NS - 1)
        def _():
            pltpu.sync_copy(zbuf.at[pl.ds(0, STRIPE)],
                            acc.at[pl.ds(base, STRIPE)])

        @pl.when(s == NS - 1)
        def _():
            pltpu.sync_copy(zbuf, acc.at[pl.ds(base, LAST_STRIPE)])

        plsc.subcore_barrier()
        cb = pl.multiple_of(w * CPW, 8)
        pltpu.sync_copy(col_hbm.at[pl.ds(cb, CPW)], col_all)
        pltpu.sync_copy(ew_hbm.at[pl.ds(cb, CPW)], ew_all)

        def grp(i, _):
            for j in range(8):
                g = i * 8 + j
                pltpu.async_copy(ew_all.at[g], acc.at[col_all.at[g]], sem,
                                 add=True)
            for j in range(8):
                g = i * 8 + j
                pltpu.make_async_copy(ew_all.at[g], acc.at[col_all.at[g]],
                                      sem).wait()
            return 0

        lax.fori_loop(0, CPW // 8, grp, 0)
        plsc.subcore_barrier()

        @pl.when(s < NS - 1)
        def _():
            pltpu.sync_copy(acc.at[pl.ds(base, STRIPE)],
                            out_hbm.at[c, pl.ds(base, STRIPE)])

        @pl.when(s == NS - 1)
        def _():
            pltpu.sync_copy(acc.at[pl.ds(base, LAST_STRIPE)],
                            out_hbm.at[c, pl.ds(base, LAST_STRIPE)])

    return k


def _make_conv_kernel(width):
    mesh = plsc.VectorSubcoreMesh(core_axis_name="c", subcore_axis_name="s")

    @functools.partial(
        pl.kernel,
        out_type=jax.ShapeDtypeStruct((NC, N, width), jnp.float32),
        mesh=mesh,
        compiler_params=_SC_PARAMS,
        scratch_types=[
            pltpu.VMEM((CPW0, CH), jnp.int32),
            pltpu.VMEM((CPW0, CH), jnp.int32),
            pltpu.VMEM((CPW0, CH), jnp.float32),
            pltpu.VMEM((CH, width), jnp.float32),
            pltpu.VMEM((CH, width), jnp.float32),
            pltpu.VMEM_SHARED((N, width), jnp.float32),
            pltpu.VMEM_SHARED((N, width), jnp.float32),
            pltpu.SemaphoreType.DMA,
            pltpu.SemaphoreType.DMA,
            pltpu.SemaphoreType.DMA,
            pltpu.SemaphoreType.DMA,
        ],
    )
    def k(row_hbm, col_hbm, ew_hbm, xws_hbm, out_hbm,
          row_all, col_all, ew_all, rows0, rows1, acc, xws_s, g0, g1, s0, s1):
        c = lax.axis_index("c")
        s = lax.axis_index("s")
        w = c * NS + s
        base = pl.multiple_of(s * STRIPE, 8)

        @pl.when(s < NS - 1)
        def _():
            pltpu.sync_copy(xws_hbm.at[pl.ds(base, STRIPE)],
                            xws_s.at[pl.ds(base, STRIPE)])

        @pl.when(s == NS - 1)
        def _():
            pltpu.sync_copy(xws_hbm.at[pl.ds(base, LAST_STRIPE)],
                            xws_s.at[pl.ds(base, LAST_STRIPE)])

        _zero_rows(rows0, CH, width)
        _zero_acc_stripe(acc, rows0, s)
        plsc.subcore_barrier()

        def run(cpw, cbase):
            pltpu.sync_copy(row_hbm.at[pl.ds(cbase, cpw)],
                            row_all.at[pl.ds(0, cpw)])
            pltpu.sync_copy(col_hbm.at[pl.ds(cbase, cpw)],
                            col_all.at[pl.ds(0, cpw)])
            pltpu.sync_copy(ew_hbm.at[pl.ds(cbase, cpw)],
                            ew_all.at[pl.ds(0, cpw)])
            pltpu.async_copy(xws_s.at[row_all.at[0]], rows0, g0)

            def pair(i, _):
                for j in range(2):
                    g = i * 2 + j
                    if j == 0:
                        cur, gc, sc_ = rows0, g0, s0
                        nxt, gn, sn_ = rows1, g1, s1
                    else:
                        cur, gc, sc_ = rows1, g1, s1
                        nxt, gn, sn_ = rows0, g0, s0
                    pltpu.make_async_copy(xws_s.at[row_all.at[g]], cur,
                                          gc).wait()

                    @pl.when(g >= 1)
                    def _():
                        pltpu.make_async_copy(nxt, acc.at[col_all.at[g - 1]],
                                              sn_).wait()

                    @pl.when(g + 1 < cpw)
                    def _():
                        pltpu.async_copy(xws_s.at[row_all.at[g + 1]], nxt,
                                         gn)

                    @plsc.parallel_loop(0, CH, step=1, unroll=8)
                    def _(k2):
                        f = plsc.load_gather(
                            ew_all, [jnp.full((16,), g, jnp.int32),
                                     jnp.full((16,), k2, jnp.int32)])
                        for p in range(width // 16):
                            v = cur[k2, pl.ds(p * 16, 16)]
                            cur[k2, pl.ds(p * 16, 16)] = v * f
                    pltpu.async_copy(cur, acc.at[col_all.at[g]], sc_,
                                     add=True)
                return 0

            lax.fori_loop(0, cpw // 2, pair, 0)
            pltpu.make_async_copy(rows1, acc.at[col_all.at[cpw - 1]],
                                  s1).wait()

        @pl.when(c == 0)
        def _():
            run(CPW0, pl.multiple_of(s * CPW0, 8))

        @pl.when(c == 1)
        def _():
            run(CPW1, pl.multiple_of(NS * CPW0 + s * CPW1, 8))

        plsc.subcore_barrier()
        _copy_out_stripe(acc, out_hbm, c, s)

    return k


_deg_kernel = _make_deg_kernel()
_conv64 = _make_conv_kernel(H1)
_conv32 = _make_conv_kernel(H2)


def _tc1_body(degp_ref, x_ref, w1_ref, dinv_ref, xws1_ref):
    deg = 1.0 + degp_ref[0] + degp_ref[1]
    dinv = lax.rsqrt(deg)
    dinv_ref[...] = dinv
    xw = jnp.dot(x_ref[...], w1_ref[...], preferred_element_type=jnp.float32)
    xws1_ref[...] = xw * dinv


def _tc2_body(s1_ref, xws1_ref, dinv_ref, b1_ref, w2_ref, xws2_ref):
    dinv = dinv_ref[...]
    pre = dinv * (s1_ref[0] + s1_ref[1] + xws1_ref[...]) + b1_ref[...]
    h1 = jnp.maximum(pre, 0.0)
    xw2 = jnp.dot(h1, w2_ref[...], preferred_element_type=jnp.float32)
    xws2_ref[...] = xw2 * dinv


WIN = 1000
NWIN = N // WIN


def _tc3_body(s2_ref, xws2_ref, dinv_ref, b2_ref, batch_ref, bnds_ref,
              wl_ref, bl_ref, out_ref, pooled):
    dinv = dinv_ref[...]
    pre = dinv * (s2_ref[0] + s2_ref[1] + xws2_ref[...]) + b2_ref[...]
    h2 = jnp.maximum(pre, 0.0)
    batch = batch_ref[...]
    pooled[...] = jnp.full((G, H2), -jnp.inf, jnp.float32)
    for wdw in range(NWIN):
        rows = h2[wdw * WIN:(wdw + 1) * WIN]
        bwin = batch[wdw * WIN:(wdw + 1) * WIN]
        g0 = bnds_ref[wdw]
        g1 = bnds_ref[NWIN + wdw]

        def seg(g, _):
            m = bwin == g
            v = jnp.where(m, rows, -jnp.inf)
            cur = pooled[pl.ds(g, 1), :]
            pooled[pl.ds(g, 1), :] = jnp.maximum(
                cur, jnp.max(v, axis=0, keepdims=True))
            return 0

        lax.fori_loop(g0, g1 + 1, seg, 0)
    out_ref[...] = jnp.dot(pooled[...], wl_ref[...],
                           preferred_element_type=jnp.float32) + bl_ref[...]


def kernel(x, edge_index, edge_weight, batch, W1, b1, W2, b2, Wl, bl):
    pad = E_PAD - E
    idt = edge_index.dtype
    row2d = jnp.concatenate(
        [edge_index[0], jnp.zeros((pad,), idt)]).reshape(NCHP, CH)
    col2d = jnp.concatenate(
        [edge_index[1], jnp.zeros((pad,), idt)]).reshape(NCHP, CH)
    ew2d = jnp.concatenate(
        [edge_weight, jnp.zeros((pad,), edge_weight.dtype)]).reshape(NCHP, CH)

    degp = _deg_kernel(col2d, ew2d).reshape(NC, N, 1)

    dinv, xws1 = pl.pallas_call(
        _tc1_body,
        out_shape=(jax.ShapeDtypeStruct((N, 1), jnp.float32),
                   jax.ShapeDtypeStruct((N, H1), jnp.float32)),
    )(degp, x, W1)

    s1 = _conv64(row2d, col2d, ew2d, xws1)

    xws2 = pl.pallas_call(
        _tc2_body,
        out_shape=jax.ShapeDtypeStruct((N, H2), jnp.float32),
    )(s1, xws1, dinv, b1.reshape(1, H1), W2)

    s2 = _conv32(row2d, col2d, ew2d, xws2)

    bnds = jnp.concatenate([batch[0::WIN], batch[WIN - 1::WIN]])

    out = pl.pallas_call(
        _tc3_body,
        out_shape=jax.ShapeDtypeStruct((G, 4), jnp.float32),
        in_specs=[
            pl.BlockSpec(), pl.BlockSpec(), pl.BlockSpec(), pl.BlockSpec(),
            pl.BlockSpec(), pl.BlockSpec(memory_space=pltpu.SMEM),
            pl.BlockSpec(), pl.BlockSpec()],
        scratch_shapes=[pltpu.VMEM((G, H2), jnp.float32)],
    )(s2, xws2, dinv, b2.reshape(1, H2), batch.reshape(N, 1),
      bnds.astype(jnp.int32), Wl, bl.reshape(1, 4))

    return out
